# baseline (speedup 1.0000x reference)
import jax
import jax.numpy as jnp
from jax.experimental import pallas as pl

_N_FIELDS, _N_NODES, _D_FEAT = 2, 10000, 512
_N_CHUNKS = 8
_CHUNK_ROWS = _N_FIELDS * _N_NODES // _N_CHUNKS


def _copy_body(in0_ref, in1_ref, out_ref):
    out_ref[0] = in0_ref[0]
    out_ref[1] = in1_ref[0]


def kernel(old_g_nodes, new_g_nodes, time_map_nodes, weight, bias):
    x = new_g_nodes.reshape(_N_CHUNKS, _CHUNK_ROWS, _D_FEAT)
    out = pl.pallas_call(
        _copy_body,
        grid=(_N_CHUNKS // 2,),
        in_specs=[
            pl.BlockSpec((1, _CHUNK_ROWS, _D_FEAT), lambda i: (2 * i, 0, 0)),
            pl.BlockSpec((1, _CHUNK_ROWS, _D_FEAT), lambda i: (2 * i + 1, 0, 0)),
        ],
        out_specs=pl.BlockSpec((2, _CHUNK_ROWS, _D_FEAT), lambda i: (i, 0, 0)),
        out_shape=jax.ShapeDtypeStruct(
            (_N_CHUNKS, _CHUNK_ROWS, _D_FEAT), jnp.float32
        ),
    )(x, x)
    return out.reshape(_N_FIELDS, _N_NODES, _D_FEAT)


# speedup vs baseline: 3.7094x; 3.7094x over previous
import jax
import jax.numpy as jnp
from jax.experimental import pallas as pl

_N_FIELDS, _N_NODES, _D_FEAT = 2, 10000, 512
_N_CHUNKS = 10
_CHUNK_ROWS = _N_FIELDS * _N_NODES // _N_CHUNKS


def _copy_body(in0_ref, in1_ref, out_ref):
    out_ref[0] = in0_ref[0]
    out_ref[1] = in1_ref[0]


def kernel(old_g_nodes, new_g_nodes, time_map_nodes, weight, bias):
    x = new_g_nodes.reshape(_N_CHUNKS, _CHUNK_ROWS, _D_FEAT)
    out = pl.pallas_call(
        _copy_body,
        grid=(_N_CHUNKS // 2,),
        in_specs=[
            pl.BlockSpec((1, _CHUNK_ROWS, _D_FEAT), lambda i: (2 * i, 0, 0)),
            pl.BlockSpec((1, _CHUNK_ROWS, _D_FEAT), lambda i: (2 * i + 1, 0, 0)),
        ],
        out_specs=pl.BlockSpec((2, _CHUNK_ROWS, _D_FEAT), lambda i: (i, 0, 0)),
        out_shape=jax.ShapeDtypeStruct(
            (_N_CHUNKS, _CHUNK_ROWS, _D_FEAT), jnp.float32
        ),
    )(x, x)
    return out.reshape(_N_FIELDS, _N_NODES, _D_FEAT)


# speedup vs baseline: 3.7345x; 1.0068x over previous
import jax
import jax.numpy as jnp
from jax.experimental import pallas as pl

_N_FIELDS, _N_NODES, _D_FEAT = 2, 10000, 512
_BLOCK_ROWS = 5000


def _copy_body(src_ref, out_ref):
    out_ref[...] = src_ref[...]


def kernel(old_g_nodes, new_g_nodes, time_map_nodes, weight, bias):
    rows = _N_FIELDS * _N_NODES
    x = new_g_nodes.reshape(rows, _D_FEAT)
    out = pl.pallas_call(
        _copy_body,
        grid=(rows // _BLOCK_ROWS,),
        in_specs=[pl.BlockSpec((_BLOCK_ROWS, _D_FEAT), lambda i: (i, 0))],
        out_specs=pl.BlockSpec((_BLOCK_ROWS, _D_FEAT), lambda i: (i, 0)),
        out_shape=jax.ShapeDtypeStruct((rows, _D_FEAT), jnp.float32),
    )(x)
    return out.reshape(_N_FIELDS, _N_NODES, _D_FEAT)
